# baseline (speedup 1.0000x reference)
import dataclasses
import functools

import jax
import jax.numpy as jnp
from jax import lax
from jax.experimental import pallas as pl
from jax.experimental.pallas import tpu as pltpu
from jax.experimental.pallas import tpu_sc as plsc

N = 10000
E = 320000
D = 128
NC = 2
NS = 16
EC = E // NC
EW = EC // NS
W = 128
CH = 8
NCH = 10
NWIN = NCH * CH
EWP = NWIN * W
ELAST = E - (NC * NS - 1) * EWP
EP = NC * NS * EWP
NDUM = 1024
NACC = N + NDUM
ZCH = 696
RCH = 640
NP = NS * RCH

_mesh = plsc.VectorSubcoreMesh(core_axis_name="c", subcore_axis_name="s")

_sc_params = pltpu.CompilerParams()
if "needs_layout_passes" in pltpu.CompilerParams.__dataclass_fields__:
    _sc_params = dataclasses.replace(_sc_params, needs_layout_passes=False)


def _degrees(ei):
    @functools.partial(
        pl.kernel,
        out_type=(
            jax.ShapeDtypeStruct((NC, 2, NP), jnp.float32),
            jax.ShapeDtypeStruct((NC * NS * EWP,), jnp.int32),
            jax.ShapeDtypeStruct((NC * NS * EWP,), jnp.int32),
        ),
        mesh=_mesh,
        compiler_params=_sc_params,
        scratch_types=[
            pltpu.VMEM_SHARED((2, NS, NP), jnp.float32),
            pltpu.VMEM((NP,), jnp.float32),
            pltpu.VMEM((NP,), jnp.float32),
            pltpu.VMEM((2, EWP), jnp.int32),
            pltpu.VMEM((EWP - ELAST,), jnp.int32),
            pltpu.VMEM((NS, RCH), jnp.float32),
            pltpu.VMEM((RCH,), jnp.float32),
        ],
    )
    def k(ei_hbm, deg_hbm, src4_hbm, dst4_hbm,
          stage_sh, hs, hd, ebuf, pbuf, red, outv):
        c = lax.axis_index("c")
        s = lax.axis_index("s")
        wid = c * NS + s
        last = wid == NC * NS - 1
        ones = jnp.ones((16,), jnp.float32)
        zeros = jnp.zeros((16,), jnp.float32)

        @pl.loop(0, NP, step=16)
        def _(i):
            hs[pl.ds(i, 16)] = zeros
            hd[pl.ds(i, 16)] = zeros

        base = wid * EWP

        @pl.when(jnp.logical_not(last))
        def _():
            pltpu.sync_copy(ei_hbm.at[pl.ds(0, 2), pl.ds(base, EWP)], ebuf)

        @pl.when(last)
        def _():
            pltpu.sync_copy(ei_hbm.at[pl.ds(0, 2), pl.ds(base, ELAST)],
                            ebuf.at[pl.ds(0, 2), pl.ds(0, ELAST)])

        cnt = jnp.where(last, ELAST, EWP)

        @pl.loop(0, cnt, step=16)
        def _(e):
            plsc.addupdate_scatter(hs, [ebuf[0, pl.ds(e, 16)]], ones)
            plsc.addupdate_scatter(hd, [ebuf[1, pl.ds(e, 16)]], ones)

        @pl.when(jnp.logical_not(last))
        def _():
            pltpu.sync_copy(ebuf.at[0], src4_hbm.at[pl.ds(base, EWP)])
            pltpu.sync_copy(ebuf.at[1], dst4_hbm.at[pl.ds(base, EWP)])

        @pl.when(last)
        def _():
            pltpu.sync_copy(ebuf.at[0, pl.ds(0, ELAST)],
                            src4_hbm.at[pl.ds(base, ELAST)])
            pltpu.sync_copy(ebuf.at[1, pl.ds(0, ELAST)],
                            dst4_hbm.at[pl.ds(base, ELAST)])

            @pl.loop(0, EWP - ELAST, step=16)
            def _(i):
                pbuf[pl.ds(i, 16)] = lax.iota(jnp.int32, 16) + i

            pltpu.sync_copy(pbuf, src4_hbm.at[pl.ds(base + ELAST, EWP - ELAST)])

            @pl.loop(0, EWP - ELAST, step=16)
            def _(i):
                v = lax.iota(jnp.int32, 16) + i
                pbuf[pl.ds(i, 16)] = jnp.bitwise_and(v, NDUM - 1) + N

            pltpu.sync_copy(pbuf, dst4_hbm.at[pl.ds(base + ELAST, EWP - ELAST)])

        pltpu.sync_copy(hs, stage_sh.at[0, s])
        pltpu.sync_copy(hd, stage_sh.at[1, s])
        plsc.subcore_barrier()

        el0 = s * RCH
        for which in range(2):
            pltpu.sync_copy(stage_sh.at[which, :, pl.ds(el0, RCH)], red)

            @pl.loop(0, RCH, step=16)
            def _(i):
                acc = red[0, pl.ds(i, 16)]
                for t in range(1, NS):
                    acc = acc + red[t, pl.ds(i, 16)]
                outv[pl.ds(i, 16)] = acc

            pltpu.sync_copy(outv, deg_hbm.at[c, which, pl.ds(el0, RCH)])

    return k(ei)


def _edge_pass(h, src4, dst4):
    @functools.partial(
        pl.kernel,
        out_type=jax.ShapeDtypeStruct((NC, N, D), jnp.float32),
        mesh=_mesh,
        scratch_types=[
            pltpu.VMEM_SHARED((NACC, D), jnp.float32),
            pltpu.VMEM((2, CH, W), jnp.int32),
            pltpu.VMEM((2, CH, W), jnp.int32),
            pltpu.VMEM((W, D), jnp.float32),
            pltpu.VMEM((W, D), jnp.float32),
            pltpu.SemaphoreType.DMA,
            pltpu.SemaphoreType.DMA,
            pltpu.SemaphoreType.DMA,
        ],
    )
    def k(h_hbm, src_hbm, dst_hbm, out_hbm, acc_sh, sidxb, didxb,
          rows0, rows1, semi, sem0, sem1):
        c = lax.axis_index("c")
        s = lax.axis_index("s")
        wid = c * NS + s

        pltpu.async_copy(src_hbm.at[wid, 0], sidxb.at[0], semi)
        pltpu.async_copy(dst_hbm.at[wid, 0], didxb.at[0], semi)

        zeros = jnp.zeros((16,), jnp.float32)

        @pl.loop(0, W)
        def _(r):
            @pl.loop(0, D, step=16)
            def _(col):
                rows0[r, pl.ds(col, 16)] = zeros

        row0z = jnp.minimum(s * ZCH, NACC - 6 * W)
        for j in range(6):
            pltpu.sync_copy(rows0, acc_sh.at[pl.ds(row0z + j * W, W)])

        pltpu.make_async_copy(src_hbm.at[wid, 0], sidxb.at[0], semi).wait()
        pltpu.make_async_copy(dst_hbm.at[wid, 0], didxb.at[0], semi).wait()
        plsc.subcore_barrier()

        pltpu.async_copy(src_hbm.at[wid, 1], sidxb.at[1], semi)
        pltpu.async_copy(dst_hbm.at[wid, 1], didxb.at[1], semi)
        pltpu.async_copy(h_hbm.at[sidxb.at[0, 0]], rows0, sem0)
        pltpu.async_copy(h_hbm.at[sidxb.at[0, 1]], rows1, sem1)

        @pl.loop(0, NCH)
        def _(ci):
            b = ci % 2
            nb = 1 - b
            for wl in range(CH):
                rbuf = rows0 if wl % 2 == 0 else rows1
                sem = sem0 if wl % 2 == 0 else sem1
                pltpu.make_async_copy(h_hbm.at[sidxb.at[b, wl]], rbuf, sem).wait()
                pltpu.sync_copy(rbuf, acc_sh.at[didxb.at[b, wl]], add=True)
                if wl < CH - 2:
                    pltpu.async_copy(h_hbm.at[sidxb.at[b, wl + 2]], rbuf, sem)
                elif wl == CH - 2:
                    @pl.when(ci < NCH - 1)
                    def _():
                        pltpu.make_async_copy(
                            src_hbm.at[wid, 0], sidxb.at[0], semi).wait()
                        pltpu.make_async_copy(
                            dst_hbm.at[wid, 0], didxb.at[0], semi).wait()
                        pltpu.async_copy(h_hbm.at[sidxb.at[nb, 0]], rbuf, sem)
                else:
                    @pl.when(ci < NCH - 1)
                    def _():
                        pltpu.async_copy(h_hbm.at[sidxb.at[nb, 1]], rbuf, sem)

                    @pl.when(ci < NCH - 2)
                    def _():
                        pltpu.async_copy(src_hbm.at[wid, ci + 2], sidxb.at[b], semi)
                        pltpu.async_copy(dst_hbm.at[wid, ci + 2], didxb.at[b], semi)

        plsc.subcore_barrier()
        row0o = jnp.minimum(s * RCH, N - RCH)
        pltpu.sync_copy(acc_sh.at[pl.ds(row0o, RCH)], out_hbm.at[c, pl.ds(row0o, RCH)])

    return k(h, src4, dst4)


_R = 1000


def _prep(features, ns):
    def body(f_ref, ns_ref, h1n_ref):
        h1n_ref[...] = f_ref[...] * ns_ref[...]

    return pl.pallas_call(
        body,
        grid=(N // _R,),
        in_specs=[
            pl.BlockSpec((_R, D), lambda i: (i, 0)),
            pl.BlockSpec((_R, 1), lambda i: (i, 0)),
        ],
        out_specs=pl.BlockSpec((_R, D), lambda i: (i, 0)),
        out_shape=jax.ShapeDtypeStruct((N, D), jnp.float32),
    )(features, ns)


def _dense(p, nd, Wm, b, ns=None, relu=False):
    def body(*refs):
        if ns is not None:
            p_ref, nd_ref, w_ref, b_ref, ns_ref, o_ref = refs
        else:
            p_ref, nd_ref, w_ref, b_ref, o_ref = refs
        agg = (p_ref[0] + p_ref[1]) * nd_ref[...]
        h = jnp.dot(agg, w_ref[...], preferred_element_type=jnp.float32)
        h = h + b_ref[...]
        if relu:
            h = jnp.maximum(h, 0.0)
        if ns is not None:
            h = h * ns_ref[...]
        o_ref[...] = h

    in_specs = [
        pl.BlockSpec((NC, _R, D), lambda i: (0, i, 0)),
        pl.BlockSpec((_R, 1), lambda i: (i, 0)),
        pl.BlockSpec((D, D), lambda i: (0, 0)),
        pl.BlockSpec((1, D), lambda i: (0, 0)),
    ]
    args = [p, nd, Wm, b.reshape(1, D)]
    if ns is not None:
        in_specs.append(pl.BlockSpec((_R, 1), lambda i: (i, 0)))
        args.append(ns)

    return pl.pallas_call(
        body,
        grid=(N // _R,),
        in_specs=in_specs,
        out_specs=pl.BlockSpec((_R, D), lambda i: (i, 0)),
        out_shape=jax.ShapeDtypeStruct((N, D), jnp.float32),
    )(*args)


def kernel(features, edge_index, W1, b1, W2, b2):
    ei = edge_index.astype(jnp.int32)
    degp, src4, dst4 = _degrees(ei)
    src4 = src4.reshape(NC * NS, NCH, CH, W)
    dst4 = dst4.reshape(NC * NS, NCH, CH, W)
    dsrc = (degp[0, 0, :N] + degp[1, 0, :N])[:, None]
    ddst = (degp[0, 1, :N] + degp[1, 1, :N])[:, None]
    ns = jnp.where(dsrc > 0, jax.lax.rsqrt(jnp.maximum(dsrc, 1.0)), 0.0)
    nd = jnp.where(ddst > 0, jax.lax.rsqrt(jnp.maximum(ddst, 1.0)), 0.0)
    h1n = _prep(features, ns)
    p1 = _edge_pass(h1n, src4, dst4)
    h2n = _dense(p1, nd, W1, b1, ns=ns, relu=True)
    p2 = _edge_pass(h2n, src4, dst4)
    return _dense(p2, nd, W2, b2)


# speedup vs baseline: 1.0218x; 1.0218x over previous
import dataclasses
import functools

import jax
import jax.numpy as jnp
from jax import lax
from jax.experimental import pallas as pl
from jax.experimental.pallas import tpu as pltpu
from jax.experimental.pallas import tpu_sc as plsc

N = 10000
E = 320000
D = 128
NC = 2
NS = 16
EC = E // NC
EW = EC // NS
W = 128
CH = 16
NCH = 5
NWIN = NCH * CH
EWP = NWIN * W
ELAST = E - (NC * NS - 1) * EWP
EP = NC * NS * EWP
NDUM = 1024
NACC = N + NDUM
ZCH = 696
RCH = 640
NP = NS * RCH

_mesh = plsc.VectorSubcoreMesh(core_axis_name="c", subcore_axis_name="s")

_sc_params = pltpu.CompilerParams()
if "needs_layout_passes" in pltpu.CompilerParams.__dataclass_fields__:
    _sc_params = dataclasses.replace(_sc_params, needs_layout_passes=False)


def _degrees(ei):
    @functools.partial(
        pl.kernel,
        out_type=(
            jax.ShapeDtypeStruct((NC, 2, NP), jnp.float32),
            jax.ShapeDtypeStruct((NC * NS * EWP,), jnp.int32),
            jax.ShapeDtypeStruct((NC * NS * EWP,), jnp.int32),
        ),
        mesh=_mesh,
        compiler_params=_sc_params,
        scratch_types=[
            pltpu.VMEM_SHARED((2, NS, NP), jnp.float32),
            pltpu.VMEM((NP,), jnp.float32),
            pltpu.VMEM((NP,), jnp.float32),
            pltpu.VMEM((2, EWP), jnp.int32),
            pltpu.VMEM((EWP - ELAST,), jnp.int32),
            pltpu.VMEM((NS, RCH), jnp.float32),
            pltpu.VMEM((RCH,), jnp.float32),
        ],
    )
    def k(ei_hbm, deg_hbm, src4_hbm, dst4_hbm,
          stage_sh, hs, hd, ebuf, pbuf, red, outv):
        c = lax.axis_index("c")
        s = lax.axis_index("s")
        wid = c * NS + s
        last = wid == NC * NS - 1
        ones = jnp.ones((16,), jnp.float32)
        zeros = jnp.zeros((16,), jnp.float32)

        @pl.loop(0, NP, step=16)
        def _(i):
            hs[pl.ds(i, 16)] = zeros
            hd[pl.ds(i, 16)] = zeros

        base = wid * EWP

        @pl.when(jnp.logical_not(last))
        def _():
            pltpu.sync_copy(ei_hbm.at[pl.ds(0, 2), pl.ds(base, EWP)], ebuf)

        @pl.when(last)
        def _():
            pltpu.sync_copy(ei_hbm.at[pl.ds(0, 2), pl.ds(base, ELAST)],
                            ebuf.at[pl.ds(0, 2), pl.ds(0, ELAST)])

        cnt = jnp.where(last, ELAST, EWP)

        @pl.loop(0, cnt, step=16)
        def _(e):
            plsc.addupdate_scatter(hs, [ebuf[0, pl.ds(e, 16)]], ones)
            plsc.addupdate_scatter(hd, [ebuf[1, pl.ds(e, 16)]], ones)

        @pl.when(jnp.logical_not(last))
        def _():
            pltpu.sync_copy(ebuf.at[0], src4_hbm.at[pl.ds(base, EWP)])
            pltpu.sync_copy(ebuf.at[1], dst4_hbm.at[pl.ds(base, EWP)])

        @pl.when(last)
        def _():
            pltpu.sync_copy(ebuf.at[0, pl.ds(0, ELAST)],
                            src4_hbm.at[pl.ds(base, ELAST)])
            pltpu.sync_copy(ebuf.at[1, pl.ds(0, ELAST)],
                            dst4_hbm.at[pl.ds(base, ELAST)])

            @pl.loop(0, EWP - ELAST, step=16)
            def _(i):
                pbuf[pl.ds(i, 16)] = lax.iota(jnp.int32, 16) + i

            pltpu.sync_copy(pbuf, src4_hbm.at[pl.ds(base + ELAST, EWP - ELAST)])

            @pl.loop(0, EWP - ELAST, step=16)
            def _(i):
                v = lax.iota(jnp.int32, 16) + i
                pbuf[pl.ds(i, 16)] = jnp.bitwise_and(v, NDUM - 1) + N

            pltpu.sync_copy(pbuf, dst4_hbm.at[pl.ds(base + ELAST, EWP - ELAST)])

        pltpu.sync_copy(hs, stage_sh.at[0, s])
        pltpu.sync_copy(hd, stage_sh.at[1, s])
        plsc.subcore_barrier()

        el0 = s * RCH
        for which in range(2):
            pltpu.sync_copy(stage_sh.at[which, :, pl.ds(el0, RCH)], red)

            @pl.loop(0, RCH, step=16)
            def _(i):
                acc = red[0, pl.ds(i, 16)]
                for t in range(1, NS):
                    acc = acc + red[t, pl.ds(i, 16)]
                outv[pl.ds(i, 16)] = acc

            pltpu.sync_copy(outv, deg_hbm.at[c, which, pl.ds(el0, RCH)])

    return k(ei)


def _edge_pass(h, src4, dst4):
    @functools.partial(
        pl.kernel,
        out_type=jax.ShapeDtypeStruct((NC, N, D), jnp.float32),
        mesh=_mesh,
        scratch_types=[
            pltpu.VMEM_SHARED((NACC, D), jnp.float32),
            pltpu.VMEM((2, CH, W), jnp.int32),
            pltpu.VMEM((2, CH, W), jnp.int32),
            pltpu.VMEM((W, D), jnp.float32),
            pltpu.VMEM((W, D), jnp.float32),
            pltpu.SemaphoreType.DMA,
            pltpu.SemaphoreType.DMA,
            pltpu.SemaphoreType.DMA,
        ],
    )
    def k(h_hbm, src_hbm, dst_hbm, out_hbm, acc_sh, sidxb, didxb,
          rows0, rows1, semi, sem0, sem1):
        c = lax.axis_index("c")
        s = lax.axis_index("s")
        wid = c * NS + s

        pltpu.async_copy(src_hbm.at[wid, 0], sidxb.at[0], semi)
        pltpu.async_copy(dst_hbm.at[wid, 0], didxb.at[0], semi)

        zeros = jnp.zeros((16,), jnp.float32)

        @pl.loop(0, W)
        def _(r):
            @pl.loop(0, D, step=16)
            def _(col):
                rows0[r, pl.ds(col, 16)] = zeros

        row0z = jnp.minimum(s * ZCH, NACC - 6 * W)
        for j in range(6):
            pltpu.sync_copy(rows0, acc_sh.at[pl.ds(row0z + j * W, W)])

        pltpu.make_async_copy(src_hbm.at[wid, 0], sidxb.at[0], semi).wait()
        pltpu.make_async_copy(dst_hbm.at[wid, 0], didxb.at[0], semi).wait()
        plsc.subcore_barrier()

        pltpu.async_copy(src_hbm.at[wid, 1], sidxb.at[1], semi)
        pltpu.async_copy(dst_hbm.at[wid, 1], didxb.at[1], semi)
        pltpu.async_copy(h_hbm.at[sidxb.at[0, 0]], rows0, sem0)
        pltpu.async_copy(h_hbm.at[sidxb.at[0, 1]], rows1, sem1)

        @pl.loop(0, NCH)
        def _(ci):
            b = ci % 2
            nb = 1 - b
            for wl in range(CH):
                rbuf = rows0 if wl % 2 == 0 else rows1
                sem = sem0 if wl % 2 == 0 else sem1
                pltpu.make_async_copy(h_hbm.at[sidxb.at[b, wl]], rbuf, sem).wait()
                pltpu.sync_copy(rbuf, acc_sh.at[didxb.at[b, wl]], add=True)
                if wl < CH - 2:
                    pltpu.async_copy(h_hbm.at[sidxb.at[b, wl + 2]], rbuf, sem)
                elif wl == CH - 2:
                    @pl.when(ci < NCH - 1)
                    def _():
                        pltpu.make_async_copy(
                            src_hbm.at[wid, 0], sidxb.at[0], semi).wait()
                        pltpu.make_async_copy(
                            dst_hbm.at[wid, 0], didxb.at[0], semi).wait()
                        pltpu.async_copy(h_hbm.at[sidxb.at[nb, 0]], rbuf, sem)
                else:
                    @pl.when(ci < NCH - 1)
                    def _():
                        pltpu.async_copy(h_hbm.at[sidxb.at[nb, 1]], rbuf, sem)

                    @pl.when(ci < NCH - 2)
                    def _():
                        pltpu.async_copy(src_hbm.at[wid, ci + 2], sidxb.at[b], semi)
                        pltpu.async_copy(dst_hbm.at[wid, ci + 2], didxb.at[b], semi)

        plsc.subcore_barrier()
        row0o = jnp.minimum(s * RCH, N - RCH)
        pltpu.sync_copy(acc_sh.at[pl.ds(row0o, RCH)], out_hbm.at[c, pl.ds(row0o, RCH)])

    return k(h, src4, dst4)


_R = 2000


def _prep(features, ns):
    def body(f_ref, ns_ref, h1n_ref):
        h1n_ref[...] = f_ref[...] * ns_ref[...]

    return pl.pallas_call(
        body,
        grid=(N // _R,),
        in_specs=[
            pl.BlockSpec((_R, D), lambda i: (i, 0)),
            pl.BlockSpec((_R, 1), lambda i: (i, 0)),
        ],
        out_specs=pl.BlockSpec((_R, D), lambda i: (i, 0)),
        out_shape=jax.ShapeDtypeStruct((N, D), jnp.float32),
    )(features, ns)


def _dense(p, nd, Wm, b, ns=None, relu=False):
    def body(*refs):
        if ns is not None:
            p_ref, nd_ref, w_ref, b_ref, ns_ref, o_ref = refs
        else:
            p_ref, nd_ref, w_ref, b_ref, o_ref = refs
        agg = (p_ref[0] + p_ref[1]) * nd_ref[...]
        h = jnp.dot(agg, w_ref[...], preferred_element_type=jnp.float32)
        h = h + b_ref[...]
        if relu:
            h = jnp.maximum(h, 0.0)
        if ns is not None:
            h = h * ns_ref[...]
        o_ref[...] = h

    in_specs = [
        pl.BlockSpec((NC, _R, D), lambda i: (0, i, 0)),
        pl.BlockSpec((_R, 1), lambda i: (i, 0)),
        pl.BlockSpec((D, D), lambda i: (0, 0)),
        pl.BlockSpec((1, D), lambda i: (0, 0)),
    ]
    args = [p, nd, Wm, b.reshape(1, D)]
    if ns is not None:
        in_specs.append(pl.BlockSpec((_R, 1), lambda i: (i, 0)))
        args.append(ns)

    return pl.pallas_call(
        body,
        grid=(N // _R,),
        in_specs=in_specs,
        out_specs=pl.BlockSpec((_R, D), lambda i: (i, 0)),
        out_shape=jax.ShapeDtypeStruct((N, D), jnp.float32),
    )(*args)


def kernel(features, edge_index, W1, b1, W2, b2):
    ei = edge_index.astype(jnp.int32)
    degp, src4, dst4 = _degrees(ei)
    src4 = src4.reshape(NC * NS, NCH, CH, W)
    dst4 = dst4.reshape(NC * NS, NCH, CH, W)
    dsrc = (degp[0, 0, :N] + degp[1, 0, :N])[:, None]
    ddst = (degp[0, 1, :N] + degp[1, 1, :N])[:, None]
    ns = jnp.where(dsrc > 0, jax.lax.rsqrt(jnp.maximum(dsrc, 1.0)), 0.0)
    nd = jnp.where(ddst > 0, jax.lax.rsqrt(jnp.maximum(ddst, 1.0)), 0.0)
    h1n = _prep(features, ns)
    p1 = _edge_pass(h1n, src4, dst4)
    h2n = _dense(p1, nd, W1, b1, ns=ns, relu=True)
    p2 = _edge_pass(h2n, src4, dst4)
    return _dense(p2, nd, W2, b2)


# speedup vs baseline: 1.0234x; 1.0016x over previous
import dataclasses
import functools

import jax
import jax.numpy as jnp
from jax import lax
from jax.experimental import pallas as pl
from jax.experimental.pallas import tpu as pltpu
from jax.experimental.pallas import tpu_sc as plsc

N = 10000
E = 320000
D = 128
NC = 2
NS = 16
EC = E // NC
EW = EC // NS
W = 128
CH = 8
NCH = 10
NWIN = NCH * CH
EWP = NWIN * W
ELAST = E - (NC * NS - 1) * EWP
EP = NC * NS * EWP
NDUM = 1024
NACC = N + NDUM
ZCH = 696
RCH = 640
NP = NS * RCH

_mesh = plsc.VectorSubcoreMesh(core_axis_name="c", subcore_axis_name="s")

_sc_params = pltpu.CompilerParams()
if "needs_layout_passes" in pltpu.CompilerParams.__dataclass_fields__:
    _sc_params = dataclasses.replace(_sc_params, needs_layout_passes=False)


def _degrees(ei):
    @functools.partial(
        pl.kernel,
        out_type=(
            jax.ShapeDtypeStruct((NC, 2, NP), jnp.float32),
            jax.ShapeDtypeStruct((NC * NS * EWP,), jnp.int32),
            jax.ShapeDtypeStruct((NC * NS * EWP,), jnp.int32),
        ),
        mesh=_mesh,
        compiler_params=_sc_params,
        scratch_types=[
            pltpu.VMEM_SHARED((2, NS, NP), jnp.float32),
            pltpu.VMEM((NP,), jnp.float32),
            pltpu.VMEM((NP,), jnp.float32),
            pltpu.VMEM((2, EWP), jnp.int32),
            pltpu.VMEM((EWP - ELAST,), jnp.int32),
            pltpu.VMEM((NS, RCH), jnp.float32),
            pltpu.VMEM((RCH,), jnp.float32),
        ],
    )
    def k(ei_hbm, deg_hbm, src4_hbm, dst4_hbm,
          stage_sh, hs, hd, ebuf, pbuf, red, outv):
        c = lax.axis_index("c")
        s = lax.axis_index("s")
        wid = c * NS + s
        last = wid == NC * NS - 1
        ones = jnp.ones((16,), jnp.float32)
        zeros = jnp.zeros((16,), jnp.float32)

        @pl.loop(0, NP, step=16)
        def _(i):
            hs[pl.ds(i, 16)] = zeros
            hd[pl.ds(i, 16)] = zeros

        base = wid * EWP

        @pl.when(jnp.logical_not(last))
        def _():
            pltpu.sync_copy(ei_hbm.at[pl.ds(0, 2), pl.ds(base, EWP)], ebuf)

        @pl.when(last)
        def _():
            pltpu.sync_copy(ei_hbm.at[pl.ds(0, 2), pl.ds(base, ELAST)],
                            ebuf.at[pl.ds(0, 2), pl.ds(0, ELAST)])

        cnt = jnp.where(last, ELAST, EWP)

        @pl.loop(0, cnt, step=16)
        def _(e):
            plsc.addupdate_scatter(hs, [ebuf[0, pl.ds(e, 16)]], ones)
            plsc.addupdate_scatter(hd, [ebuf[1, pl.ds(e, 16)]], ones)

        @pl.when(jnp.logical_not(last))
        def _():
            pltpu.sync_copy(ebuf.at[0], src4_hbm.at[pl.ds(base, EWP)])
            pltpu.sync_copy(ebuf.at[1], dst4_hbm.at[pl.ds(base, EWP)])

        @pl.when(last)
        def _():
            pltpu.sync_copy(ebuf.at[0, pl.ds(0, ELAST)],
                            src4_hbm.at[pl.ds(base, ELAST)])
            pltpu.sync_copy(ebuf.at[1, pl.ds(0, ELAST)],
                            dst4_hbm.at[pl.ds(base, ELAST)])

            @pl.loop(0, EWP - ELAST, step=16)
            def _(i):
                pbuf[pl.ds(i, 16)] = lax.iota(jnp.int32, 16) + i

            pltpu.sync_copy(pbuf, src4_hbm.at[pl.ds(base + ELAST, EWP - ELAST)])

            @pl.loop(0, EWP - ELAST, step=16)
            def _(i):
                v = lax.iota(jnp.int32, 16) + i
                pbuf[pl.ds(i, 16)] = jnp.bitwise_and(v, NDUM - 1) + N

            pltpu.sync_copy(pbuf, dst4_hbm.at[pl.ds(base + ELAST, EWP - ELAST)])

        pltpu.sync_copy(hs, stage_sh.at[0, s])
        pltpu.sync_copy(hd, stage_sh.at[1, s])
        plsc.subcore_barrier()

        el0 = s * RCH
        for which in range(2):
            pltpu.sync_copy(stage_sh.at[which, :, pl.ds(el0, RCH)], red)

            @pl.loop(0, RCH, step=16)
            def _(i):
                acc = red[0, pl.ds(i, 16)]
                for t in range(1, NS):
                    acc = acc + red[t, pl.ds(i, 16)]
                outv[pl.ds(i, 16)] = acc

            pltpu.sync_copy(outv, deg_hbm.at[c, which, pl.ds(el0, RCH)])

    return k(ei)


def _edge_pass(h, src4, dst4):
    @functools.partial(
        pl.kernel,
        out_type=jax.ShapeDtypeStruct((NC, N, D), jnp.float32),
        mesh=_mesh,
        scratch_types=[
            pltpu.VMEM_SHARED((NACC, D), jnp.float32),
            pltpu.VMEM((2, CH, W), jnp.int32),
            pltpu.VMEM((2, CH, W), jnp.int32),
            pltpu.VMEM((W, D), jnp.float32),
            pltpu.VMEM((W, D), jnp.float32),
            pltpu.SemaphoreType.DMA,
            pltpu.SemaphoreType.DMA,
            pltpu.SemaphoreType.DMA,
        ],
    )
    def k(h_hbm, src_hbm, dst_hbm, out_hbm, acc_sh, sidxb, didxb,
          rows0, rows1, semi, sem0, sem1):
        c = lax.axis_index("c")
        s = lax.axis_index("s")
        wid = c * NS + s

        pltpu.async_copy(src_hbm.at[wid, 0], sidxb.at[0], semi)
        pltpu.async_copy(dst_hbm.at[wid, 0], didxb.at[0], semi)

        zeros = jnp.zeros((16,), jnp.float32)

        @pl.loop(0, W)
        def _(r):
            @pl.loop(0, D, step=16)
            def _(col):
                rows0[r, pl.ds(col, 16)] = zeros

        row0z = jnp.minimum(s * ZCH, NACC - 6 * W)
        for j in range(6):
            pltpu.sync_copy(rows0, acc_sh.at[pl.ds(row0z + j * W, W)])

        pltpu.make_async_copy(src_hbm.at[wid, 0], sidxb.at[0], semi).wait()
        pltpu.make_async_copy(dst_hbm.at[wid, 0], didxb.at[0], semi).wait()
        plsc.subcore_barrier()

        pltpu.async_copy(src_hbm.at[wid, 1], sidxb.at[1], semi)
        pltpu.async_copy(dst_hbm.at[wid, 1], didxb.at[1], semi)
        pltpu.async_copy(h_hbm.at[sidxb.at[0, 0]], rows0, sem0)
        pltpu.async_copy(h_hbm.at[sidxb.at[0, 1]], rows1, sem1)

        @pl.loop(0, NCH)
        def _(ci):
            b = ci % 2
            nb = 1 - b
            for wl in range(CH):
                rbuf = rows0 if wl % 2 == 0 else rows1
                sem = sem0 if wl % 2 == 0 else sem1
                pltpu.make_async_copy(h_hbm.at[sidxb.at[b, wl]], rbuf, sem).wait()
                pltpu.sync_copy(rbuf, acc_sh.at[didxb.at[b, wl]], add=True)
                if wl < CH - 2:
                    pltpu.async_copy(h_hbm.at[sidxb.at[b, wl + 2]], rbuf, sem)
                elif wl == CH - 2:
                    @pl.when(ci < NCH - 1)
                    def _():
                        pltpu.make_async_copy(
                            src_hbm.at[wid, 0], sidxb.at[0], semi).wait()
                        pltpu.make_async_copy(
                            dst_hbm.at[wid, 0], didxb.at[0], semi).wait()
                        pltpu.async_copy(h_hbm.at[sidxb.at[nb, 0]], rbuf, sem)
                else:
                    @pl.when(ci < NCH - 1)
                    def _():
                        pltpu.async_copy(h_hbm.at[sidxb.at[nb, 1]], rbuf, sem)

                    @pl.when(ci < NCH - 2)
                    def _():
                        pltpu.async_copy(src_hbm.at[wid, ci + 2], sidxb.at[b], semi)
                        pltpu.async_copy(dst_hbm.at[wid, ci + 2], didxb.at[b], semi)

        plsc.subcore_barrier()
        row0o = jnp.minimum(s * RCH, N - RCH)
        pltpu.sync_copy(acc_sh.at[pl.ds(row0o, RCH)], out_hbm.at[c, pl.ds(row0o, RCH)])

    return k(h, src4, dst4)


_R = 2000


def _prep(features, ns):
    def body(f_ref, ns_ref, h1n_ref):
        h1n_ref[...] = f_ref[...] * ns_ref[...]

    return pl.pallas_call(
        body,
        grid=(N // _R,),
        in_specs=[
            pl.BlockSpec((_R, D), lambda i: (i, 0)),
            pl.BlockSpec((_R, 1), lambda i: (i, 0)),
        ],
        out_specs=pl.BlockSpec((_R, D), lambda i: (i, 0)),
        out_shape=jax.ShapeDtypeStruct((N, D), jnp.float32),
    )(features, ns)


def _dense(p, nd, Wm, b, ns=None, relu=False):
    def body(*refs):
        if ns is not None:
            p_ref, nd_ref, w_ref, b_ref, ns_ref, o_ref = refs
        else:
            p_ref, nd_ref, w_ref, b_ref, o_ref = refs
        agg = (p_ref[0] + p_ref[1]) * nd_ref[...]
        h = jnp.dot(agg, w_ref[...], preferred_element_type=jnp.float32)
        h = h + b_ref[...]
        if relu:
            h = jnp.maximum(h, 0.0)
        if ns is not None:
            h = h * ns_ref[...]
        o_ref[...] = h

    in_specs = [
        pl.BlockSpec((NC, _R, D), lambda i: (0, i, 0)),
        pl.BlockSpec((_R, 1), lambda i: (i, 0)),
        pl.BlockSpec((D, D), lambda i: (0, 0)),
        pl.BlockSpec((1, D), lambda i: (0, 0)),
    ]
    args = [p, nd, Wm, b.reshape(1, D)]
    if ns is not None:
        in_specs.append(pl.BlockSpec((_R, 1), lambda i: (i, 0)))
        args.append(ns)

    return pl.pallas_call(
        body,
        grid=(N // _R,),
        in_specs=in_specs,
        out_specs=pl.BlockSpec((_R, D), lambda i: (i, 0)),
        out_shape=jax.ShapeDtypeStruct((N, D), jnp.float32),
    )(*args)


def kernel(features, edge_index, W1, b1, W2, b2):
    ei = edge_index.astype(jnp.int32)
    degp, src4, dst4 = _degrees(ei)
    src4 = src4.reshape(NC * NS, NCH, CH, W)
    dst4 = dst4.reshape(NC * NS, NCH, CH, W)
    dsrc = (degp[0, 0, :N] + degp[1, 0, :N])[:, None]
    ddst = (degp[0, 1, :N] + degp[1, 1, :N])[:, None]
    ns = jnp.where(dsrc > 0, jax.lax.rsqrt(jnp.maximum(dsrc, 1.0)), 0.0)
    nd = jnp.where(ddst > 0, jax.lax.rsqrt(jnp.maximum(ddst, 1.0)), 0.0)
    h1n = _prep(features, ns)
    p1 = _edge_pass(h1n, src4, dst4)
    h2n = _dense(p1, nd, W1, b1, ns=ns, relu=True)
    p2 = _edge_pass(h2n, src4, dst4)
    return _dense(p2, nd, W2, b2)
